# baseline (speedup 1.0000x reference)
import functools

import numpy as np
import jax
import jax.numpy as jnp
from jax import lax
from jax.experimental import pallas as pl
from jax.experimental.pallas import tpu as pltpu
from jax.experimental.pallas import tpu_sc as plsc

_PREC = jax.lax.Precision.HIGHEST
_BOUNDS = ((0, 7), (6, 14), (13, 21), (20, 27))


def _colsel() -> np.ndarray:
    s = np.zeros((56, 4), np.float32)
    for j, (cs, ce) in enumerate(_BOUNDS):
        for pw in range(cs, ce):
            s[2 * pw, j] = 1.0 / (ce - cs)
    return s


def _rowsel() -> np.ndarray:
    s = np.zeros((224, 56), np.float32)
    for ow in range(56):
        s[4 * ow, ow] = 1.0
    return s


def _shift(m, s):
    if s == 0:
        return m
    if s > 0:
        return jnp.concatenate(
            [m[:, s:], jnp.zeros((16, s), jnp.float32)], axis=1)
    return jnp.concatenate(
        [jnp.zeros((16, -s), jnp.float32), m[:, :s]], axis=1)


def _taps(out1):
    acc = jnp.zeros((16, 224), jnp.float32)
    for kx in range(7):
        acc = acc + _shift(out1[16 * kx:16 * (kx + 1), :], kx - 3)
    return acc


def _dot(a, b):
    return jax.lax.dot_general(a, b, (((1,), (0,)), ((), ())),
                               preferred_element_type=jnp.float32,
                               precision=_PREC)


def _dot_bf(a, b):
    return jax.lax.dot_general(a, b, (((1,), (0,)), ((), ())),
                               preferred_element_type=jnp.float32,
                               precision=jax.lax.Precision.DEFAULT)


def _conv_row(wstk32, whi, R, Rlo):
    m1 = _dot_bf(wstk32, R)
    m2 = _dot_bf(whi, Rlo)
    return m1[:112, :] + m1[112:, :] + m2


def _tc_body(x_hbm, wstk_ref, whi_ref, cb_ref, rsel_ref, csel_ref, w1_ref, b1_ref,
             w2_ref, b2_ref, temp_ref, logits_ref,
             xbuf, rlo_s, t_all, conv_s, mp_s, sems):
    b = pl.program_id(0)
    nb = pl.num_programs(0)

    def copy_img(img, slot):
        return [pltpu.make_async_copy(x_hbm.at[img, c],
                                      xbuf.at[slot, :, c, :],
                                      sems.at[slot, c % 4])
                for c in range(96)]

    @pl.when(b == 0)
    def _():
        for cp in copy_img(0, 0):
            cp.start()

    @pl.when(b + 1 < nb)
    def _():
        for cp in copy_img(b + 1, (b + 1) % 2):
            cp.start()

    slot = jax.lax.rem(b, 2)
    for cp in copy_img(b, slot):
        cp.wait()

    wstk = wstk_ref[...]
    whi = whi_ref[...]

    def cast_rows(lo):
        v = xbuf[slot, lo]
        hi = v.astype(jnp.bfloat16)
        rlo_s[lo] = (v - hi.astype(jnp.float32)).astype(jnp.bfloat16)

    cast_rows(pl.ds(0, 4))
    R0 = xbuf[slot, 0:4, :, :].reshape(4 * 96, 224)
    L0 = rlo_s[0:4, :, :].reshape(4 * 96, 224)
    t_all[0] = _taps(_conv_row(wstk[:, 3 * 96:], whi[:, 3 * 96:], R0, L0))

    def oh_body(oh, _):
        cast_rows(pl.ds(4 * oh, 4))
        R = xbuf[slot, pl.ds(4 * oh - 3, 7), :, :].reshape(7 * 96, 224)
        Rlo = rlo_s[pl.ds(4 * oh - 3, 7), :, :].reshape(7 * 96, 224)
        t_all[oh] = _taps(_conv_row(wstk, whi, R, Rlo))
        return _

    jax.lax.fori_loop(1, 56, oh_body, None)

    tt = t_all[...].reshape(56 * 16, 224)
    cr = _dot(tt, rsel_ref[...]).reshape(56, 16, 56)
    conv_s[...] = jnp.maximum(cr + cb_ref[...], 0.0)

    zcol = jnp.zeros((16, 1), jnp.float32)

    def mp_body(ph, _):
        m = jnp.maximum(jnp.maximum(conv_s[2 * ph], conv_s[2 * ph + 1]),
                        conv_s[2 * ph + 2])
        sh1 = jnp.concatenate([m[:, 1:], zcol], axis=1)
        sh2 = jnp.concatenate([m[:, 2:], zcol, zcol], axis=1)
        mp_s[ph] = jnp.maximum(jnp.maximum(m, sh1), sh2)
        return _

    jax.lax.fori_loop(0, 27, mp_body, None)

    csel = csel_ref[...]
    fcols = []
    for (rs, re) in _BOUNDS:
        rsum = mp_s[rs]
        for r in range(rs + 1, re):
            rsum = rsum + mp_s[r]
        fcols.append(_dot(rsum, csel) * (1.0 / (re - rs)))
    F = jnp.concatenate(fcols, axis=1)
    feat = jnp.concatenate([F[c:c + 1, :] for c in range(16)], axis=1)

    h1 = jnp.maximum(_dot(feat, w1_ref[...]) + b1_ref[...], 0.0)
    logits = _dot(h1, w2_ref[...]) + b2_ref[...]
    t = jnp.clip(temp_ref[0, 0], 0.5, 5.0)
    logits_ref[0, 0, :] = (logits / t)[0]


def _maxall(v):
    return plsc.cummax(lax.rev(plsc.cummax(v), (0,)))


def _sc_route_body(logits_hbm, gates_hbm, idx_hbm, row_v, gout_v, iout_v):
    wid = lax.axis_index("s") * 2 + lax.axis_index("c")

    @pl.when(wid < 16)
    def _():
        pltpu.sync_copy(logits_hbm.at[wid], row_v)
        lr = row_v[...]
        iota = lax.iota(jnp.int32, 16)
        m1 = _maxall(lr)
        i1 = -_maxall(jnp.where(lr == m1, -iota, -16))
        masked = jnp.where(iota == i1, -jnp.inf, lr)
        m2 = _maxall(masked)
        i2 = -_maxall(jnp.where(masked == m2, -iota, -16))
        e2 = jnp.exp(m2 - m1)
        s1 = 1.0 / (1.0 + e2)
        s2 = e2 / (1.0 + e2)
        den = s1 + s2 + 1e-10
        gout_v[...] = jnp.where(iota == i1, s1 / den,
                                jnp.where(iota == i2, s2 / den, 0.0))
        iout_v[...] = jnp.where(iota == 0, i1,
                                jnp.where(iota == 1, i2, 0))
        pltpu.sync_copy(gout_v, gates_hbm.at[wid])
        pltpu.sync_copy(iout_v, idx_hbm.at[wid])


_sc_route = functools.partial(
    pl.kernel,
    out_type=[
        jax.ShapeDtypeStruct((16, 16), jnp.float32),
        jax.ShapeDtypeStruct((16, 16), jnp.int32),
    ],
    mesh=plsc.VectorSubcoreMesh(core_axis_name="c", subcore_axis_name="s"),
    compiler_params=pltpu.CompilerParams(needs_layout_passes=False),
    scratch_types=[
        pltpu.VMEM((16,), jnp.float32),
        pltpu.VMEM((16,), jnp.float32),
        pltpu.VMEM((16,), jnp.int32),
    ],
)(_sc_route_body)


def kernel(x, conv_w, conv_b, fc1_w, fc1_b, fc2_w, fc2_b, temperature):
    B = x.shape[0]
    wall = jnp.transpose(conv_w, (3, 0, 2, 1)).reshape(112, 672)
    whi = wall.astype(jnp.bfloat16)
    wlo = (wall - whi.astype(jnp.float32)).astype(jnp.bfloat16)
    wstk = jnp.concatenate([whi, wlo], axis=0).astype(jnp.float32)
    cb = conv_b.reshape(16, 1)
    rsel = jnp.asarray(_rowsel())
    csel = jnp.asarray(_colsel())
    w1 = fc1_w.T
    b1 = fc1_b.reshape(1, 64)
    w2 = fc2_w.T
    b2 = fc2_b.reshape(1, 16)
    temp = temperature.reshape(1, 1)

    rep = lambda *shape: pl.BlockSpec(shape, lambda b: (0,) * len(shape))
    (logits3,) = pl.pallas_call(
        _tc_body,
        grid=(B,),
        in_specs=[
            pl.BlockSpec(memory_space=pl.ANY),
            rep(224, 672),
            rep(112, 672),
            rep(16, 1),
            rep(224, 56),
            rep(56, 4),
            rep(256, 64),
            rep(1, 64),
            rep(64, 16),
            rep(1, 16),
            pl.BlockSpec(memory_space=pltpu.SMEM),
        ],
        out_specs=[
            pl.BlockSpec((1, 1, 16), lambda b: (b, 0, 0)),
        ],
        out_shape=[
            jax.ShapeDtypeStruct((B, 1, 16), jnp.float32),
        ],
        scratch_shapes=[
            pltpu.VMEM((2, 224, 96, 224), jnp.float32),
            pltpu.VMEM((224, 96, 224), jnp.bfloat16),
            pltpu.VMEM((56, 16, 224), jnp.float32),
            pltpu.VMEM((56, 16, 56), jnp.float32),
            pltpu.VMEM((27, 16, 56), jnp.float32),
            pltpu.SemaphoreType.DMA((2, 4)),
        ],
        compiler_params=pltpu.CompilerParams(
            dimension_semantics=("arbitrary",)),
    )(x, wstk, whi, cb, rsel, csel, w1, b1, w2, b2, temp)

    gate_logits = logits3[:, 0, :]
    gates, idxpad = _sc_route(gate_logits)
    top_k_indices = idxpad[:, :2]
    return gates, top_k_indices, gate_logits


# speedup vs baseline: 2.6513x; 2.6513x over previous
import functools

import numpy as np
import jax
import jax.numpy as jnp
from jax import lax
from jax.experimental import pallas as pl
from jax.experimental.pallas import tpu as pltpu
from jax.experimental.pallas import tpu_sc as plsc

_PREC = jax.lax.Precision.HIGHEST
_BOUNDS = ((0, 7), (6, 14), (13, 21), (20, 27))


def _colsel() -> np.ndarray:
    s = np.zeros((56, 4), np.float32)
    for j, (cs, ce) in enumerate(_BOUNDS):
        for pw in range(cs, ce):
            s[2 * pw, j] = 1.0 / (ce - cs)
    return s


def _rowsel() -> np.ndarray:
    s = np.zeros((224, 56), np.float32)
    for ow in range(56):
        s[4 * ow, ow] = 1.0
    return s


def _shift(m, s):
    if s == 0:
        return m
    if s > 0:
        return jnp.concatenate(
            [m[:, s:], jnp.zeros((16, s), jnp.float32)], axis=1)
    return jnp.concatenate(
        [jnp.zeros((16, -s), jnp.float32), m[:, :s]], axis=1)


def _taps(out1):
    acc = jnp.zeros((16, 224), jnp.float32)
    for kx in range(7):
        acc = acc + _shift(out1[16 * kx:16 * (kx + 1), :], kx - 3)
    return acc


def _dot(a, b):
    return jax.lax.dot_general(a, b, (((1,), (0,)), ((), ())),
                               preferred_element_type=jnp.float32,
                               precision=_PREC)


def _dot_bf(a, b):
    return jax.lax.dot_general(a, b, (((1,), (0,)), ((), ())),
                               preferred_element_type=jnp.float32,
                               precision=jax.lax.Precision.DEFAULT)


def _conv_row(wstk, whi, R):
    rhi = R.astype(jnp.bfloat16)
    rlo = (R - rhi.astype(jnp.float32)).astype(jnp.bfloat16)
    m1 = _dot_bf(wstk, rhi)
    m2 = _dot_bf(whi, rlo)
    return m1[:112, :] + m1[112:, :] + m2


def _tc_body(x_hbm, wstk_ref, whi_ref, cb_ref, rsel_ref, csel_ref, w1_ref, b1_ref,
             w2_ref, b2_ref, temp_ref, logits_ref,
             xbuf, t_all, conv_s, mp_s, sems):
    b = pl.program_id(0)
    nb = pl.num_programs(0)

    def copy_img(img, slot):
        return [pltpu.make_async_copy(x_hbm.at[img, c],
                                      xbuf.at[slot, :, c, :],
                                      sems.at[slot, c % 4])
                for c in range(96)]

    @pl.when(b == 0)
    def _():
        for cp in copy_img(0, 0):
            cp.start()

    @pl.when(b + 1 < nb)
    def _():
        for cp in copy_img(b + 1, (b + 1) % 2):
            cp.start()

    slot = jax.lax.rem(b, 2)
    for cp in copy_img(b, slot):
        cp.wait()

    wstk = wstk_ref[...]
    whi = whi_ref[...]

    R0 = xbuf[slot, 0:4, :, :].reshape(4 * 96, 224)
    t_all[0] = _taps(_conv_row(wstk[:, 3 * 96:], whi[:, 3 * 96:], R0))

    for oh in range(1, 56):
        R = xbuf[slot, pl.ds(4 * oh - 3, 7), :, :].reshape(7 * 96, 224)
        t_all[oh] = _taps(_conv_row(wstk, whi, R))

    tt = t_all[...].reshape(56 * 16, 224)
    cr = _dot(tt, rsel_ref[...]).reshape(56, 16, 56)
    conv_s[...] = jnp.maximum(cr + cb_ref[...], 0.0)

    zcol = jnp.zeros((16, 1), jnp.float32)

    for ph in range(27):
        m = jnp.maximum(jnp.maximum(conv_s[2 * ph], conv_s[2 * ph + 1]),
                        conv_s[2 * ph + 2])
        sh1 = jnp.concatenate([m[:, 1:], zcol], axis=1)
        sh2 = jnp.concatenate([m[:, 2:], zcol, zcol], axis=1)
        mp_s[ph] = jnp.maximum(jnp.maximum(m, sh1), sh2)

    csel = csel_ref[...]
    fcols = []
    for (rs, re) in _BOUNDS:
        rsum = mp_s[rs]
        for r in range(rs + 1, re):
            rsum = rsum + mp_s[r]
        fcols.append(_dot(rsum, csel) * (1.0 / (re - rs)))
    F = jnp.concatenate(fcols, axis=1)
    feat = jnp.concatenate([F[c:c + 1, :] for c in range(16)], axis=1)

    h1 = jnp.maximum(_dot(feat, w1_ref[...]) + b1_ref[...], 0.0)
    logits = _dot(h1, w2_ref[...]) + b2_ref[...]
    t = jnp.clip(temp_ref[0, 0], 0.5, 5.0)
    logits_ref[0, 0, :] = (logits / t)[0]


def _maxall(v):
    return plsc.cummax(lax.rev(plsc.cummax(v), (0,)))


def _sc_route_body(logits_hbm, gates_hbm, idx_hbm, row_v, gout_v, iout_v):
    wid = lax.axis_index("s") * 2 + lax.axis_index("c")

    @pl.when(wid < 16)
    def _():
        pltpu.sync_copy(logits_hbm.at[wid], row_v)
        lr = row_v[...]
        iota = lax.iota(jnp.int32, 16)
        m1 = _maxall(lr)
        i1 = -_maxall(jnp.where(lr == m1, -iota, -16))
        masked = jnp.where(iota == i1, -jnp.inf, lr)
        m2 = _maxall(masked)
        i2 = -_maxall(jnp.where(masked == m2, -iota, -16))
        e2 = jnp.exp(m2 - m1)
        s1 = 1.0 / (1.0 + e2)
        s2 = e2 / (1.0 + e2)
        den = s1 + s2 + 1e-10
        gout_v[...] = jnp.where(iota == i1, s1 / den,
                                jnp.where(iota == i2, s2 / den, 0.0))
        iout_v[...] = jnp.where(iota == 0, i1,
                                jnp.where(iota == 1, i2, 0))
        pltpu.sync_copy(gout_v, gates_hbm.at[wid])
        pltpu.sync_copy(iout_v, idx_hbm.at[wid])


_sc_route = functools.partial(
    pl.kernel,
    out_type=[
        jax.ShapeDtypeStruct((16, 16), jnp.float32),
        jax.ShapeDtypeStruct((16, 16), jnp.int32),
    ],
    mesh=plsc.VectorSubcoreMesh(core_axis_name="c", subcore_axis_name="s"),
    compiler_params=pltpu.CompilerParams(needs_layout_passes=False),
    scratch_types=[
        pltpu.VMEM((16,), jnp.float32),
        pltpu.VMEM((16,), jnp.float32),
        pltpu.VMEM((16,), jnp.int32),
    ],
)(_sc_route_body)


def kernel(x, conv_w, conv_b, fc1_w, fc1_b, fc2_w, fc2_b, temperature):
    B = x.shape[0]
    wall = jnp.transpose(conv_w, (3, 0, 2, 1)).reshape(112, 672)
    whi = wall.astype(jnp.bfloat16)
    wlo = (wall - whi.astype(jnp.float32)).astype(jnp.bfloat16)
    wstk = jnp.concatenate([whi, wlo], axis=0)
    cb = conv_b.reshape(16, 1)
    rsel = jnp.asarray(_rowsel())
    csel = jnp.asarray(_colsel())
    w1 = fc1_w.T
    b1 = fc1_b.reshape(1, 64)
    w2 = fc2_w.T
    b2 = fc2_b.reshape(1, 16)
    temp = temperature.reshape(1, 1)

    rep = lambda *shape: pl.BlockSpec(shape, lambda b: (0,) * len(shape))
    (logits3,) = pl.pallas_call(
        _tc_body,
        grid=(B,),
        in_specs=[
            pl.BlockSpec(memory_space=pl.ANY),
            rep(224, 672),
            rep(112, 672),
            rep(16, 1),
            rep(224, 56),
            rep(56, 4),
            rep(256, 64),
            rep(1, 64),
            rep(64, 16),
            rep(1, 16),
            pl.BlockSpec(memory_space=pltpu.SMEM),
        ],
        out_specs=[
            pl.BlockSpec((1, 1, 16), lambda b: (b, 0, 0)),
        ],
        out_shape=[
            jax.ShapeDtypeStruct((B, 1, 16), jnp.float32),
        ],
        scratch_shapes=[
            pltpu.VMEM((2, 224, 96, 224), jnp.float32),
            pltpu.VMEM((56, 16, 224), jnp.float32),
            pltpu.VMEM((56, 16, 56), jnp.float32),
            pltpu.VMEM((27, 16, 56), jnp.float32),
            pltpu.SemaphoreType.DMA((2, 4)),
        ],
        compiler_params=pltpu.CompilerParams(
            dimension_semantics=("arbitrary",)),
    )(x, wstk, whi, cb, rsel, csel, w1, b1, w2, b2, temp)

    gate_logits = logits3[:, 0, :]
    gates, idxpad = _sc_route(gate_logits)
    top_k_indices = idxpad[:, :2]
    return gates, top_k_indices, gate_logits
